# Initial kernel scaffold; baseline (speedup 1.0000x reference)
import functools

import jax
import jax.numpy as jnp
from jax import lax
from jax.experimental import pallas as pl
from jax.experimental.pallas import tpu as pltpu
from jax.experimental.pallas import tpu_sc as plsc

S = 10000
D = 128
N = 320000
BLK = 128
NBLK = N // BLK
NC = 2
NS = 16
NW = NC * NS
ROWS_PER_SUB = S // NS

_mesh = plsc.VectorSubcoreMesh(core_axis_name="c", subcore_axis_name="s")


@functools.partial(
    pl.kernel,
    out_type=jax.ShapeDtypeStruct((NC, S, D), jnp.float32),
    mesh=_mesh,
    scratch_types=[
        pltpu.VMEM_SHARED((S, D), jnp.float32),
        pltpu.VMEM((BLK, D), jnp.float32),
        pltpu.VMEM((1, BLK), jnp.int32),
        pltpu.VMEM((25, D), jnp.float32),
    ],
)
def _segsum_partials(v_hbm, idx_hbm, out_hbm, acc, vbuf, ibuf, zbuf):
    c = lax.axis_index("c")
    s = lax.axis_index("s")
    w = s * NC + c

    zero16 = jnp.zeros((16,), jnp.float32)
    for r in range(25):
        for k in range(D // 16):
            zbuf[r, pl.ds(16 * k, 16)] = zero16
    base = s * ROWS_PER_SUB

    def zbody(i, carry):
        pltpu.sync_copy(zbuf, acc.at[pl.ds(base + i * 25, 25)])
        return carry

    lax.fori_loop(0, ROWS_PER_SUB // 25, zbody, 0)
    plsc.subcore_barrier()

    nit = NBLK // NW + 1

    def mbody(t, carry):
        j = w + t * NW

        @pl.when(j < NBLK)
        def _():
            pltpu.sync_copy(idx_hbm.at[j], ibuf.at[0])
            pltpu.sync_copy(v_hbm.at[pl.ds(j * BLK, BLK)], vbuf)
            pltpu.sync_copy(vbuf, acc.at[ibuf.at[0]], add=True)

        return carry

    lax.fori_loop(0, nit, mbody, 0)
    plsc.subcore_barrier()

    for i in range(ROWS_PER_SUB // 125):
        off = base + i * 125
        pltpu.sync_copy(acc.at[pl.ds(off, 125)], out_hbm.at[c, pl.ds(off, 125)])


def _combine_body(u_ref, p0_ref, p1_ref, o_ref):
    o_ref[...] = u_ref[...] + p0_ref[...] + p1_ref[...]


def _combine(u, p0, p1):
    RB = 2000
    spec = pl.BlockSpec((RB, D), lambda i: (i, 0))
    return pl.pallas_call(
        _combine_body,
        out_shape=jax.ShapeDtypeStruct((S, D), jnp.float32),
        grid=(S // RB,),
        in_specs=[spec, spec, spec],
        out_specs=spec,
    )(u, p0, p1)


def kernel(u, v, batch):
    idx = batch.astype(jnp.int32).reshape(NBLK, BLK)
    p = _segsum_partials(v, idx)
    return _combine(u, p[0], p[1])


# speedup vs baseline: 4.3414x; 4.3414x over previous
import functools

import jax
import jax.numpy as jnp
from jax import lax
from jax.experimental import pallas as pl
from jax.experimental.pallas import tpu as pltpu
from jax.experimental.pallas import tpu_sc as plsc

S = 10000
D = 128
N = 320000
BLK = 128
NBLK = N // BLK
NC = 2
NS = 16
NW = NC * NS
ROWS_PER_SUB = 624
TAIL_BASE = NS * ROWS_PER_SUB
TAIL_ROWS = S - TAIL_BASE

_mesh = plsc.VectorSubcoreMesh(core_axis_name="c", subcore_axis_name="s")


@functools.partial(
    pl.kernel,
    out_type=jax.ShapeDtypeStruct((NC, S, D), jnp.float32),
    mesh=_mesh,
    scratch_types=[
        pltpu.VMEM_SHARED((S, D), jnp.float32),
        pltpu.VMEM((BLK, D), jnp.float32),
        pltpu.VMEM((1, BLK), jnp.int32),
        pltpu.VMEM((48, D), jnp.float32),
    ],
)
def _segsum_partials(v_hbm, idx_hbm, out_hbm, acc, vbuf, ibuf, zbuf):
    c = lax.axis_index("c")
    s = lax.axis_index("s")
    w = s * NC + c

    zero16 = jnp.zeros((16,), jnp.float32)
    for r in range(48):
        for k in range(D // 16):
            zbuf[r, pl.ds(16 * k, 16)] = zero16
    base = s * ROWS_PER_SUB

    def zbody(i, carry):
        pltpu.sync_copy(zbuf, acc.at[pl.ds(base + i * 48, 48)])
        return carry

    lax.fori_loop(0, ROWS_PER_SUB // 48, zbody, 0)

    @pl.when(s == NS - 1)
    def _():
        pltpu.sync_copy(zbuf.at[pl.ds(0, TAIL_ROWS)],
                        acc.at[pl.ds(TAIL_BASE, TAIL_ROWS)])

    plsc.subcore_barrier()

    nit = NBLK // NW + 1

    def mbody(t, carry):
        j = w + t * NW

        @pl.when(j < NBLK)
        def _():
            pltpu.sync_copy(idx_hbm.at[j], ibuf)
            pltpu.sync_copy(v_hbm.at[pl.ds(j * BLK, BLK)], vbuf)
            pltpu.sync_copy(vbuf, acc.at[ibuf.at[0]], add=True)

        return carry

    lax.fori_loop(0, nit, mbody, 0)
    plsc.subcore_barrier()

    pltpu.sync_copy(acc.at[pl.ds(base, ROWS_PER_SUB)],
                    out_hbm.at[c, pl.ds(base, ROWS_PER_SUB)])

    @pl.when(s == NS - 1)
    def _():
        pltpu.sync_copy(acc.at[pl.ds(TAIL_BASE, TAIL_ROWS)],
                        out_hbm.at[c, pl.ds(TAIL_BASE, TAIL_ROWS)])


def _combine_body(u_ref, p0_ref, p1_ref, o_ref):
    o_ref[...] = u_ref[...] + p0_ref[...] + p1_ref[...]


def _combine(u, p0, p1):
    RB = 2000
    spec = pl.BlockSpec((RB, D), lambda i: (i, 0))
    return pl.pallas_call(
        _combine_body,
        out_shape=jax.ShapeDtypeStruct((S, D), jnp.float32),
        grid=(S // RB,),
        in_specs=[spec, spec, spec],
        out_specs=spec,
    )(u, p0, p1)


def kernel(u, v, batch):
    idx = batch.astype(jnp.int32).reshape(NBLK, 1, BLK)
    p = _segsum_partials(v, idx)
    return _combine(u, p[0], p[1])


# speedup vs baseline: 6.9984x; 1.6120x over previous
import functools

import jax
import jax.numpy as jnp
from jax import lax
from jax.experimental import pallas as pl
from jax.experimental.pallas import tpu as pltpu
from jax.experimental.pallas import tpu_sc as plsc

S = 10000
D = 128
N = 320000
BLK = 128
NBLK = N // BLK
NC = 2
NS = 16
NW = NC * NS
BPW = NBLK // NW
EXTRA = NBLK - BPW * NW
NITER = BPW + 1
NPAIR = 40
ROWS_PER_SUB = 624
TAIL_BASE = NS * ROWS_PER_SUB
TAIL_ROWS = S - TAIL_BASE
IDX_PAD = BPW * NW + EXTRA + NITER

_mesh = plsc.VectorSubcoreMesh(core_axis_name="c", subcore_axis_name="s")


@functools.partial(
    pl.kernel,
    out_type=jax.ShapeDtypeStruct((NC, S, D), jnp.float32),
    mesh=_mesh,
    scratch_types=[
        pltpu.VMEM_SHARED((S, D), jnp.float32),
        pltpu.VMEM((2, BLK, D), jnp.float32),
        pltpu.VMEM((NITER, 1, BLK), jnp.int32),
        pltpu.VMEM((8, D), jnp.float32),
        pltpu.SemaphoreType.DMA((2,)),
    ],
)
def _segsum_partials(v_hbm, idx_hbm, out_hbm, acc, vbuf, ibuf, zbuf, gsem):
    c = lax.axis_index("c")
    s = lax.axis_index("s")
    w = s * NC + c
    start = w * BPW + jnp.minimum(w, EXTRA)
    nmine = jnp.where(w < EXTRA, BPW + 1, BPW)

    zero16 = jnp.zeros((16,), jnp.float32)
    for r in range(8):
        for k in range(D // 16):
            zbuf[r, pl.ds(16 * k, 16)] = zero16
    base = s * ROWS_PER_SUB

    def zbody(i, carry):
        pltpu.sync_copy(zbuf, acc.at[pl.ds(base + i * 8, 8)])
        return carry

    lax.fori_loop(0, ROWS_PER_SUB // 8, zbody, 0)

    @pl.when(s == NS - 1)
    def _():
        pltpu.sync_copy(zbuf, acc.at[pl.ds(TAIL_BASE, 8)])
        pltpu.sync_copy(zbuf, acc.at[pl.ds(TAIL_BASE + 8, 8)])

    pltpu.sync_copy(idx_hbm.at[pl.ds(start, NITER)], ibuf)

    def fire(t, b):
        @pl.when(t < nmine)
        def _():
            pltpu.async_copy(v_hbm.at[pl.ds((start + t) * BLK, BLK)],
                             vbuf.at[b], gsem.at[b])

    def consume(t, b):
        @pl.when(t < nmine)
        def _():
            pltpu.make_async_copy(v_hbm.at[pl.ds((start + t) * BLK, BLK)],
                                  vbuf.at[b], gsem.at[b]).wait()
            pltpu.sync_copy(vbuf.at[b], acc.at[ibuf.at[t, 0]], add=True)

    fire(0, 0)

    def mbody(i, carry):
        t0 = 2 * i
        fire(t0 + 1, 1)
        consume(t0, 0)
        fire(t0 + 2, 0)
        consume(t0 + 1, 1)
        return carry

    lax.fori_loop(0, NPAIR, mbody, 0)
    plsc.subcore_barrier()

    pltpu.sync_copy(acc.at[pl.ds(base, ROWS_PER_SUB)],
                    out_hbm.at[c, pl.ds(base, ROWS_PER_SUB)])

    @pl.when(s == NS - 1)
    def _():
        pltpu.sync_copy(acc.at[pl.ds(TAIL_BASE, TAIL_ROWS)],
                        out_hbm.at[c, pl.ds(TAIL_BASE, TAIL_ROWS)])


def _combine_body(u_ref, p0_ref, p1_ref, o_ref):
    o_ref[...] = u_ref[...] + p0_ref[...] + p1_ref[...]


def _combine(u, p0, p1):
    RB = 2000
    spec = pl.BlockSpec((RB, D), lambda i: (i, 0))
    return pl.pallas_call(
        _combine_body,
        out_shape=jax.ShapeDtypeStruct((S, D), jnp.float32),
        grid=(S // RB,),
        in_specs=[spec, spec, spec],
        out_specs=spec,
    )(u, p0, p1)


def kernel(u, v, batch):
    idx = batch.astype(jnp.int32).reshape(NBLK, BLK)
    idx = jnp.pad(idx, ((0, IDX_PAD - NBLK), (0, 0))).reshape(IDX_PAD, 1, BLK)
    p = _segsum_partials(v, idx)
    return _combine(u, p[0], p[1])
